# baseline (speedup 1.0000x reference)
import functools

import jax
import jax.numpy as jnp
from jax import lax
from jax.experimental import pallas as pl
from jax.experimental.pallas import tpu as pltpu
from jax.experimental.pallas import tpu_sc as plsc

TEMPERATURE = 0.5
MOMENTUM = 0.1
H = 32
N = 32768
A = 16
K = 64
L = 16
NSUB = 16
CHUNK = N // NSUB
NVEC = CHUNK // L

_MESH = plsc.VectorSubcoreMesh(core_axis_name="c", subcore_axis_name="s")


def _thresh_body(v_ref, out_ref):
    v = v_ref[...]
    bits = lax.bitcast_convert_type(v, jnp.uint32)
    key = jnp.where(bits >> 31 != 0, ~bits, bits | jnp.uint32(0x80000000))

    def step(i, t):
        b = (31 - i).astype(jnp.uint32)
        cand = t | (jnp.left_shift(jnp.uint32(1), b))
        cnt = jnp.sum((key >= cand).astype(jnp.int32))
        return jnp.where(cnt >= K, cand, t)

    tstar = lax.fori_loop(0, 32, step, jnp.uint32(0))
    fbits = jnp.where(tstar >> 31 != 0, tstar & jnp.uint32(0x7FFFFFFF), ~tstar)
    t_f = lax.bitcast_convert_type(fbits, jnp.float32)
    lane = lax.broadcasted_iota(jnp.int32, (8, 128), 1)
    out_ref[...] = jnp.where(lane == 0, t_f, 0.0)


def _sel_body(value_hbm, tf_hbm, oidx_hbm, ow_hbm,
              vloc, gtbuf_i, gtbuf_v, eqbuf_i, tf_v, cnts_stage, cnts_all,
              pos_v, eidx_v, eval_v, oidx_stage, ow_stage, val_stage,
              sh_counts, sh_eidx, sh_eval):
    cid = lax.axis_index("c")
    sid = lax.axis_index("s")
    lanes = lax.iota(jnp.int32, L)

    pltpu.sync_copy(tf_hbm.at[pl.ds(0, L)], tf_v)
    pltpu.sync_copy(value_hbm.at[pl.ds(sid * CHUNK, CHUNK)], vloc)
    tfv = tf_v[...]
    t_f = jnp.sum(jnp.where(lanes == 0, tfv, jnp.zeros_like(tfv)))
    tf_splat = jnp.full((L,), t_f, jnp.float32)

    def scan_vec(j, carry):
        cnt_gt, cnt_eq = carry
        v = vloc[pl.ds(j * L, L)]
        gidx = sid * CHUNK + j * L + lanes
        m_gt = v > tf_splat
        m_eq = v == tf_splat
        plsc.store_compressed(gtbuf_i.at[pl.ds(cnt_gt, L)], gidx, mask=m_gt)
        plsc.store_compressed(gtbuf_v.at[pl.ds(cnt_gt, L)], v, mask=m_gt)
        plsc.store_compressed(eqbuf_i.at[pl.ds(jnp.minimum(cnt_eq, K), L)],
                              gidx, mask=m_eq)
        cnt_gt = cnt_gt + jnp.sum(m_gt.astype(jnp.int32))
        cnt_eq = cnt_eq + jnp.sum(m_eq.astype(jnp.int32))
        return cnt_gt, cnt_eq

    cnt_gt, cnt_eq = lax.fori_loop(0, NVEC, scan_vec,
                                   (jnp.int32(0), jnp.int32(0)))

    cnts_stage[...] = jnp.where(lanes == 0, cnt_gt,
                                jnp.where(lanes == 1, cnt_eq, 0))
    pltpu.sync_copy(cnts_stage, sh_counts.at[sid])

    @pl.when(sid == 0)
    def _init_vals():
        for c in range(6):
            val_stage[pl.ds(c * L, L)] = tf_splat
        pltpu.sync_copy(val_stage, sh_eval)

    plsc.subcore_barrier()

    pltpu.sync_copy(sh_counts, cnts_all)
    zeros16 = jnp.zeros((L,), jnp.int32)
    cg = plsc.load_gather(cnts_all, [lanes, zeros16])
    ce = plsc.load_gather(cnts_all, [lanes, zeros16 + 1])
    incl_g = plsc.cumsum(cg)
    incl_e = plsc.cumsum(ce)
    excl_g = incl_g - cg
    excl_e = incl_e - ce
    n_gt_tot = jnp.max(incl_g)
    my_excl_g = jnp.sum(jnp.where(lanes == sid, excl_g, zeros16))
    my_excl_e = jnp.sum(jnp.where(lanes == sid, excl_e, zeros16))

    for c in range(4):
        li = c * L + lanes
        pos = my_excl_g + li
        ok = li < cnt_gt
        pos_v[pl.ds(c * L, L)] = jnp.where(ok, pos, K + lanes)
    pltpu.sync_copy(gtbuf_i.at[pl.ds(0, K)], sh_eidx.at[pos_v])
    pltpu.sync_copy(gtbuf_v.at[pl.ds(0, K)], sh_eval.at[pos_v])

    for c in range(4):
        li = c * L + lanes
        dest = n_gt_tot + my_excl_e + li
        ok = (li < cnt_eq) & (dest < K)
        pos_v[pl.ds(c * L, L)] = jnp.where(ok, dest, K + L + lanes)
    pltpu.sync_copy(eqbuf_i.at[pl.ds(0, K)], sh_eidx.at[pos_v])

    plsc.subcore_barrier()

    @pl.when((cid == 0) & (sid == 0))
    def _emit():
        pltpu.sync_copy(sh_eidx.at[pl.ds(0, K)], eidx_v)
        pltpu.sync_copy(sh_eval.at[pl.ds(0, K)], eval_v)
        vmax = jnp.full((L,), -jnp.inf, jnp.float32)
        for c in range(4):
            vmax = jnp.maximum(vmax, eval_v[pl.ds(c * L, L)])
        vmax_splat = jnp.full((L,), jnp.max(vmax), jnp.float32)
        s_sum = jnp.float32(0.0)
        for c in range(4):
            e = jnp.exp(TEMPERATURE * (eval_v[pl.ds(c * L, L)] - vmax_splat))
            ow_stage[pl.ds(c * L, L)] = e
            s_sum = s_sum + jnp.sum(e)
        s_splat = jnp.full((L,), s_sum, jnp.float32)
        scale = (1.0 - MOMENTUM) / (s_splat * (1.0 + 1e-9))
        zf = jnp.zeros((L,), jnp.float32)
        zi = jnp.zeros((L,), jnp.int32)
        for c in range(4):
            ow_stage[pl.ds(c * L, L)] = ow_stage[pl.ds(c * L, L)] * scale
            oidx_stage[pl.ds(c * L, L)] = eidx_v[pl.ds(c * L, L)]
        for c in range(4, 8):
            ow_stage[pl.ds(c * L, L)] = zf
            oidx_stage[pl.ds(c * L, L)] = zi
        pltpu.sync_copy(oidx_stage, oidx_hbm)
        pltpu.sync_copy(ow_stage, ow_hbm)


_sel_call = functools.partial(
    pl.kernel,
    mesh=_MESH,
    out_type=(
        jax.ShapeDtypeStruct((128,), jnp.int32),
        jax.ShapeDtypeStruct((128,), jnp.float32),
    ),
    compiler_params=pltpu.CompilerParams(
        needs_layout_passes=False, use_tc_tiling_on_sc=False
    ),
    scratch_types=[
        pltpu.VMEM((CHUNK,), jnp.float32),
        pltpu.VMEM((96,), jnp.int32),
        pltpu.VMEM((96,), jnp.float32),
        pltpu.VMEM((96,), jnp.int32),
        pltpu.VMEM((L,), jnp.float32),
        pltpu.VMEM((L,), jnp.int32),
        pltpu.VMEM((NSUB, L), jnp.int32),
        pltpu.VMEM((K,), jnp.int32),
        pltpu.VMEM((K,), jnp.int32),
        pltpu.VMEM((K,), jnp.float32),
        pltpu.VMEM((128,), jnp.int32),
        pltpu.VMEM((128,), jnp.float32),
        pltpu.VMEM((96,), jnp.float32),
        pltpu.VMEM_SHARED((NSUB, L), jnp.int32),
        pltpu.VMEM_SHARED((96,), jnp.int32),
        pltpu.VMEM_SHARED((96,), jnp.float32),
    ],
)(_sel_body)


def _tc_gather_body(eidx_ref, ew_ref, a_ref, mean_ref, out_ref):
    kk = pl.program_id(0)

    @pl.when(kk == 0)
    def _init():
        out_ref[...] = MOMENTUM * mean_ref[...]

    e = eidx_ref[kk]
    w = ew_ref[kk]
    col = e & 127
    a = a_ref[...]
    lane = lax.broadcasted_iota(jnp.int32, (H, A, 128), 2)
    sel = jnp.where(lane == col, a, 0.0)
    out_ref[...] += w * jnp.sum(sel, axis=2)


def kernel(value, actions, mean, k):
    tf_arr = pl.pallas_call(
        _thresh_body,
        out_shape=jax.ShapeDtypeStruct((8, 128), jnp.float32),
    )(value.reshape(256, 128))
    eidx, ew = _sel_call(value.reshape(N), tf_arr.reshape(1024))
    out = pl.pallas_call(
        _tc_gather_body,
        grid_spec=pltpu.PrefetchScalarGridSpec(
            num_scalar_prefetch=2,
            grid=(K,),
            in_specs=[
                pl.BlockSpec((H, A, 128),
                             lambda kk, ei, ew_: (0, 0, ei[kk] >> 7)),
                pl.BlockSpec((H, A), lambda kk, ei, ew_: (0, 0)),
            ],
            out_specs=pl.BlockSpec((H, A), lambda kk, ei, ew_: (0, 0)),
        ),
        out_shape=jax.ShapeDtypeStruct((H, A), jnp.float32),
    )(eidx, ew, actions.transpose(0, 2, 1), mean)
    return out


# speedup vs baseline: 1.5319x; 1.5319x over previous
import functools

import jax
import jax.numpy as jnp
from jax import lax
from jax.experimental import pallas as pl
from jax.experimental.pallas import tpu as pltpu
from jax.experimental.pallas import tpu_sc as plsc

TEMPERATURE = 0.5
MOMENTUM = 0.1
H = 32
N = 32768
A = 16
K = 64
L = 16
NSUB = 16
CHUNK = N // NSUB
NVEC = CHUNK // L

_MESH = plsc.VectorSubcoreMesh(core_axis_name="c", subcore_axis_name="s")


def _thresh_body(v_ref, out_ref):
    v = v_ref[...]
    bits = lax.bitcast_convert_type(v, jnp.uint32)
    key = jnp.where(bits >> 31 != 0, ~bits, bits | jnp.uint32(0x80000000))

    def step(i, t):
        b = (31 - i).astype(jnp.uint32)
        cand = t | (jnp.left_shift(jnp.uint32(1), b))
        cnt = jnp.sum((key >= cand).astype(jnp.int32))
        return jnp.where(cnt >= K, cand, t)

    tstar = lax.fori_loop(0, 32, step, jnp.uint32(0))
    fbits = jnp.where(tstar >> 31 != 0, tstar & jnp.uint32(0x7FFFFFFF), ~tstar)
    t_f = lax.bitcast_convert_type(fbits, jnp.float32)
    lane = lax.broadcasted_iota(jnp.int32, (8, 128), 1)
    out_ref[...] = jnp.where(lane == 0, t_f, 0.0)


def _sel_body(value_hbm, tf_hbm, oidx_hbm, ow_hbm,
              vloc, gtbuf_i, gtbuf_v, eqbuf_i, tf_v, cnts_stage, cnts_all,
              pos_v, eidx_v, eval_v, oidx_stage, ow_stage, val_stage,
              sh_counts, sh_eidx, sh_eval):
    cid = lax.axis_index("c")
    sid = lax.axis_index("s")
    lanes = lax.iota(jnp.int32, L)

    pltpu.sync_copy(tf_hbm.at[pl.ds(0, L)], tf_v)
    pltpu.sync_copy(value_hbm.at[pl.ds(sid * CHUNK, CHUNK)], vloc)
    tfv = tf_v[...]
    t_f = jnp.sum(jnp.where(lanes == 0, tfv, jnp.zeros_like(tfv)))
    tf_splat = jnp.full((L,), t_f, jnp.float32)

    def scan_vec(j, carry):
        cnt_gt, cnt_eq = carry
        v = vloc[pl.ds(j * L, L)]
        gidx = sid * CHUNK + j * L + lanes
        m_gt = v > tf_splat
        m_eq = v == tf_splat
        plsc.store_compressed(gtbuf_i.at[pl.ds(cnt_gt, L)], gidx, mask=m_gt)
        plsc.store_compressed(gtbuf_v.at[pl.ds(cnt_gt, L)], v, mask=m_gt)
        plsc.store_compressed(eqbuf_i.at[pl.ds(jnp.minimum(cnt_eq, K), L)],
                              gidx, mask=m_eq)
        cnt_gt = cnt_gt + jnp.sum(m_gt.astype(jnp.int32))
        cnt_eq = cnt_eq + jnp.sum(m_eq.astype(jnp.int32))
        return cnt_gt, cnt_eq

    cnt_gt, cnt_eq = lax.fori_loop(0, NVEC, scan_vec,
                                   (jnp.int32(0), jnp.int32(0)))

    cnts_stage[...] = jnp.where(lanes == 0, cnt_gt,
                                jnp.where(lanes == 1, cnt_eq, 0))
    pltpu.sync_copy(cnts_stage, sh_counts.at[sid])

    @pl.when(sid == 0)
    def _init_vals():
        for c in range(6):
            val_stage[pl.ds(c * L, L)] = tf_splat
        pltpu.sync_copy(val_stage, sh_eval)

    plsc.subcore_barrier()

    pltpu.sync_copy(sh_counts, cnts_all)
    zeros16 = jnp.zeros((L,), jnp.int32)
    cg = plsc.load_gather(cnts_all, [lanes, zeros16])
    ce = plsc.load_gather(cnts_all, [lanes, zeros16 + 1])
    incl_g = plsc.cumsum(cg)
    incl_e = plsc.cumsum(ce)
    excl_g = incl_g - cg
    excl_e = incl_e - ce
    n_gt_tot = jnp.max(incl_g)
    my_excl_g = jnp.sum(jnp.where(lanes == sid, excl_g, zeros16))
    my_excl_e = jnp.sum(jnp.where(lanes == sid, excl_e, zeros16))

    for c in range(4):
        li = c * L + lanes
        pos = my_excl_g + li
        ok = li < cnt_gt
        pos_v[pl.ds(c * L, L)] = jnp.where(ok, pos, K + lanes)
    pltpu.sync_copy(gtbuf_i.at[pl.ds(0, K)], sh_eidx.at[pos_v])
    pltpu.sync_copy(gtbuf_v.at[pl.ds(0, K)], sh_eval.at[pos_v])

    for c in range(4):
        li = c * L + lanes
        dest = n_gt_tot + my_excl_e + li
        ok = (li < cnt_eq) & (dest < K)
        pos_v[pl.ds(c * L, L)] = jnp.where(ok, dest, K + L + lanes)
    pltpu.sync_copy(eqbuf_i.at[pl.ds(0, K)], sh_eidx.at[pos_v])

    plsc.subcore_barrier()

    @pl.when((cid == 0) & (sid == 0))
    def _emit():
        pltpu.sync_copy(sh_eidx.at[pl.ds(0, K)], eidx_v)
        pltpu.sync_copy(sh_eval.at[pl.ds(0, K)], eval_v)
        vmax = jnp.full((L,), -jnp.inf, jnp.float32)
        for c in range(4):
            vmax = jnp.maximum(vmax, eval_v[pl.ds(c * L, L)])
        vmax_splat = jnp.full((L,), jnp.max(vmax), jnp.float32)
        s_sum = jnp.float32(0.0)
        for c in range(4):
            e = jnp.exp(TEMPERATURE * (eval_v[pl.ds(c * L, L)] - vmax_splat))
            ow_stage[pl.ds(c * L, L)] = e
            s_sum = s_sum + jnp.sum(e)
        s_splat = jnp.full((L,), s_sum, jnp.float32)
        scale = (1.0 - MOMENTUM) / (s_splat * (1.0 + 1e-9))
        zf = jnp.zeros((L,), jnp.float32)
        zi = jnp.zeros((L,), jnp.int32)
        for c in range(4):
            ow_stage[pl.ds(c * L, L)] = ow_stage[pl.ds(c * L, L)] * scale
            oidx_stage[pl.ds(c * L, L)] = eidx_v[pl.ds(c * L, L)]
        for c in range(4, 8):
            ow_stage[pl.ds(c * L, L)] = zf
            oidx_stage[pl.ds(c * L, L)] = zi
        pltpu.sync_copy(oidx_stage, oidx_hbm)
        pltpu.sync_copy(ow_stage, ow_hbm)


_sel_call = functools.partial(
    pl.kernel,
    mesh=_MESH,
    out_type=(
        jax.ShapeDtypeStruct((128,), jnp.int32),
        jax.ShapeDtypeStruct((128,), jnp.float32),
    ),
    compiler_params=pltpu.CompilerParams(
        needs_layout_passes=False, use_tc_tiling_on_sc=False
    ),
    scratch_types=[
        pltpu.VMEM((CHUNK,), jnp.float32),
        pltpu.VMEM((96,), jnp.int32),
        pltpu.VMEM((96,), jnp.float32),
        pltpu.VMEM((96,), jnp.int32),
        pltpu.VMEM((L,), jnp.float32),
        pltpu.VMEM((L,), jnp.int32),
        pltpu.VMEM((NSUB, L), jnp.int32),
        pltpu.VMEM((K,), jnp.int32),
        pltpu.VMEM((K,), jnp.int32),
        pltpu.VMEM((K,), jnp.float32),
        pltpu.VMEM((128,), jnp.int32),
        pltpu.VMEM((128,), jnp.float32),
        pltpu.VMEM((96,), jnp.float32),
        pltpu.VMEM_SHARED((NSUB, L), jnp.int32),
        pltpu.VMEM_SHARED((96,), jnp.int32),
        pltpu.VMEM_SHARED((96,), jnp.float32),
    ],
)(_sel_body)


def _gather_body(act_hbm, eidx_hbm, ew_hbm, out_hbm,
                 eidx_v, ew_v, rows_a, rows_b, out_stage, sem_a, sem_b):
    cid = lax.axis_index("c")
    sid = lax.axis_index("s")
    lanes = lax.iota(jnp.int32, L)
    h = cid * NSUB + sid

    pltpu.sync_copy(eidx_hbm, eidx_v)
    pltpu.sync_copy(ew_hbm, ew_v)

    evs = [eidx_v[pl.ds(c * L, L)] for c in range(4)]
    wvs = [ew_v[pl.ds(c * L, L)] for c in range(4)]
    bufs = [rows_a, rows_b]
    sems = [sem_a, sem_b]

    def issue(c):
        buf, sem = bufs[c % 2], sems[c % 2]
        return [pltpu.async_copy(
            act_hbm.at[h, :, pl.ds((evs[c][ll] >> 7) * 128, 128)],
            buf.at[ll], sem) for ll in range(L)]

    acc = jnp.zeros((L,), jnp.float32)
    pend = issue(0)
    for c in range(4):
        nxt = issue(c + 1) if c < 3 else []
        for cp in pend:
            cp.wait()
        buf = bufs[c % 2]
        for ll in range(L):
            col = plsc.load_gather(
                buf, [jnp.full((L,), ll, jnp.int32), lanes,
                      jnp.full((L,), evs[c][ll] & 127, jnp.int32)])
            acc = acc + jnp.full((L,), wvs[c][ll], jnp.float32) * col
        pend = nxt

    out_stage[pl.ds(0, L)] = acc
    pltpu.sync_copy(out_stage, out_hbm.at[pl.ds(h * 128, 128)])


_gather_call = functools.partial(
    pl.kernel,
    mesh=_MESH,
    out_type=jax.ShapeDtypeStruct((H * 128,), jnp.float32),
    compiler_params=pltpu.CompilerParams(needs_layout_passes=False),
    scratch_types=[
        pltpu.VMEM((128,), jnp.int32),
        pltpu.VMEM((128,), jnp.float32),
        pltpu.VMEM((L, A, 128), jnp.float32),
        pltpu.VMEM((L, A, 128), jnp.float32),
        pltpu.VMEM((128,), jnp.float32),
        pltpu.SemaphoreType.DMA,
        pltpu.SemaphoreType.DMA,
    ],
)(_gather_body)


def kernel(value, actions, mean, k):
    tf_arr = pl.pallas_call(
        _thresh_body,
        out_shape=jax.ShapeDtypeStruct((8, 128), jnp.float32),
    )(value.reshape(256, 128))
    eidx, ew = _sel_call(value.reshape(N), tf_arr.reshape(1024))
    wmean = _gather_call(actions.transpose(0, 2, 1), eidx, ew)
    return MOMENTUM * mean + wmean.reshape(H, 128)[:, :A]


# speedup vs baseline: 1.5327x; 1.0005x over previous
import functools

import jax
import jax.numpy as jnp
from jax import lax
from jax.experimental import pallas as pl
from jax.experimental.pallas import tpu as pltpu
from jax.experimental.pallas import tpu_sc as plsc

TEMPERATURE = 0.5
MOMENTUM = 0.1
H = 32
N = 32768
A = 16
K = 64
L = 16
NSUB = 16
CHUNK = N // NSUB
NVEC = CHUNK // L

_MESH = plsc.VectorSubcoreMesh(core_axis_name="c", subcore_axis_name="s")


def _thresh_body(v_ref, out_ref):
    v = v_ref[...]
    bits = lax.bitcast_convert_type(v, jnp.uint32)
    key = jnp.where(bits >> 31 != 0, ~bits, bits | jnp.uint32(0x80000000))

    tstar = jnp.uint32(0)
    for b in range(31, -1, -1):
        cand = tstar | jnp.uint32(1 << b)
        cnt = jnp.sum((key >= cand).astype(jnp.int32))
        tstar = jnp.where(cnt >= K, cand, tstar)
    fbits = jnp.where(tstar >> 31 != 0, tstar & jnp.uint32(0x7FFFFFFF), ~tstar)
    t_f = lax.bitcast_convert_type(fbits, jnp.float32)
    lane = lax.broadcasted_iota(jnp.int32, (8, 128), 1)
    out_ref[...] = jnp.where(lane == 0, t_f, 0.0)


def _sel_body(value_hbm, tf_hbm, oidx_hbm, ow_hbm,
              vloc, gtbuf_i, gtbuf_v, eqbuf_i, tf_v, cnts_stage, cnts_all,
              pos_v, eidx_v, eval_v, oidx_stage, ow_stage, val_stage,
              sh_counts, sh_eidx, sh_eval):
    cid = lax.axis_index("c")
    sid = lax.axis_index("s")
    lanes = lax.iota(jnp.int32, L)

    pltpu.sync_copy(tf_hbm.at[pl.ds(0, L)], tf_v)
    pltpu.sync_copy(value_hbm.at[pl.ds(sid * CHUNK, CHUNK)], vloc)
    tfv = tf_v[...]
    t_f = jnp.sum(jnp.where(lanes == 0, tfv, jnp.zeros_like(tfv)))
    tf_splat = jnp.full((L,), t_f, jnp.float32)

    def scan_vec(j, carry):
        cnt_gt, cnt_eq = carry
        for u in range(2):
            v = vloc[pl.ds((j * 2 + u) * L, L)]
            gidx = sid * CHUNK + (j * 2 + u) * L + lanes
            m_gt = v > tf_splat
            m_eq = v == tf_splat
            plsc.store_compressed(gtbuf_i.at[pl.ds(cnt_gt, L)], gidx,
                                  mask=m_gt)
            plsc.store_compressed(gtbuf_v.at[pl.ds(cnt_gt, L)], v, mask=m_gt)
            plsc.store_compressed(eqbuf_i.at[pl.ds(jnp.minimum(cnt_eq, K), L)],
                                  gidx, mask=m_eq)
            cnt_gt = cnt_gt + jnp.sum(m_gt.astype(jnp.int32))
            cnt_eq = cnt_eq + jnp.sum(m_eq.astype(jnp.int32))
        return cnt_gt, cnt_eq

    cnt_gt, cnt_eq = lax.fori_loop(0, NVEC // 2, scan_vec,
                                   (jnp.int32(0), jnp.int32(0)))

    cnts_stage[...] = jnp.where(lanes == 0, cnt_gt,
                                jnp.where(lanes == 1, cnt_eq, 0))
    pltpu.sync_copy(cnts_stage, sh_counts.at[sid])

    @pl.when(sid == 0)
    def _init_vals():
        for c in range(6):
            val_stage[pl.ds(c * L, L)] = tf_splat
        pltpu.sync_copy(val_stage, sh_eval)

    plsc.subcore_barrier()

    pltpu.sync_copy(sh_counts, cnts_all)
    zeros16 = jnp.zeros((L,), jnp.int32)
    cg = plsc.load_gather(cnts_all, [lanes, zeros16])
    ce = plsc.load_gather(cnts_all, [lanes, zeros16 + 1])
    incl_g = plsc.cumsum(cg)
    incl_e = plsc.cumsum(ce)
    excl_g = incl_g - cg
    excl_e = incl_e - ce
    n_gt_tot = jnp.max(incl_g)
    my_excl_g = jnp.sum(jnp.where(lanes == sid, excl_g, zeros16))
    my_excl_e = jnp.sum(jnp.where(lanes == sid, excl_e, zeros16))

    for c in range(4):
        li = c * L + lanes
        pos = my_excl_g + li
        ok = li < cnt_gt
        pos_v[pl.ds(c * L, L)] = jnp.where(ok, pos, K + lanes)
    pltpu.sync_copy(gtbuf_i.at[pl.ds(0, K)], sh_eidx.at[pos_v])
    pltpu.sync_copy(gtbuf_v.at[pl.ds(0, K)], sh_eval.at[pos_v])

    for c in range(4):
        li = c * L + lanes
        dest = n_gt_tot + my_excl_e + li
        ok = (li < cnt_eq) & (dest < K)
        pos_v[pl.ds(c * L, L)] = jnp.where(ok, dest, K + L + lanes)
    pltpu.sync_copy(eqbuf_i.at[pl.ds(0, K)], sh_eidx.at[pos_v])

    plsc.subcore_barrier()

    @pl.when((cid == 0) & (sid == 0))
    def _emit():
        pltpu.sync_copy(sh_eidx.at[pl.ds(0, K)], eidx_v)
        pltpu.sync_copy(sh_eval.at[pl.ds(0, K)], eval_v)
        vmax = jnp.full((L,), -jnp.inf, jnp.float32)
        for c in range(4):
            vmax = jnp.maximum(vmax, eval_v[pl.ds(c * L, L)])
        vmax_splat = jnp.full((L,), jnp.max(vmax), jnp.float32)
        s_sum = jnp.float32(0.0)
        for c in range(4):
            e = jnp.exp(TEMPERATURE * (eval_v[pl.ds(c * L, L)] - vmax_splat))
            ow_stage[pl.ds(c * L, L)] = e
            s_sum = s_sum + jnp.sum(e)
        s_splat = jnp.full((L,), s_sum, jnp.float32)
        scale = (1.0 - MOMENTUM) / (s_splat * (1.0 + 1e-9))
        zf = jnp.zeros((L,), jnp.float32)
        zi = jnp.zeros((L,), jnp.int32)
        for c in range(4):
            ow_stage[pl.ds(c * L, L)] = ow_stage[pl.ds(c * L, L)] * scale
            oidx_stage[pl.ds(c * L, L)] = eidx_v[pl.ds(c * L, L)]
        for c in range(4, 8):
            ow_stage[pl.ds(c * L, L)] = zf
            oidx_stage[pl.ds(c * L, L)] = zi
        pltpu.sync_copy(oidx_stage, oidx_hbm)
        pltpu.sync_copy(ow_stage, ow_hbm)


_sel_call = functools.partial(
    pl.kernel,
    mesh=_MESH,
    out_type=(
        jax.ShapeDtypeStruct((128,), jnp.int32),
        jax.ShapeDtypeStruct((128,), jnp.float32),
    ),
    compiler_params=pltpu.CompilerParams(
        needs_layout_passes=False, use_tc_tiling_on_sc=False
    ),
    scratch_types=[
        pltpu.VMEM((CHUNK,), jnp.float32),
        pltpu.VMEM((96,), jnp.int32),
        pltpu.VMEM((96,), jnp.float32),
        pltpu.VMEM((96,), jnp.int32),
        pltpu.VMEM((L,), jnp.float32),
        pltpu.VMEM((L,), jnp.int32),
        pltpu.VMEM((NSUB, L), jnp.int32),
        pltpu.VMEM((K,), jnp.int32),
        pltpu.VMEM((K,), jnp.int32),
        pltpu.VMEM((K,), jnp.float32),
        pltpu.VMEM((128,), jnp.int32),
        pltpu.VMEM((128,), jnp.float32),
        pltpu.VMEM((96,), jnp.float32),
        pltpu.VMEM_SHARED((NSUB, L), jnp.int32),
        pltpu.VMEM_SHARED((96,), jnp.int32),
        pltpu.VMEM_SHARED((96,), jnp.float32),
    ],
)(_sel_body)


def _gather_body(act_hbm, eidx_hbm, ew_hbm, out_hbm,
                 eidx_v, ew_v, rows_a, rows_b, out_stage, sem_a, sem_b):
    cid = lax.axis_index("c")
    sid = lax.axis_index("s")
    lanes = lax.iota(jnp.int32, L)
    h = cid * NSUB + sid

    pltpu.sync_copy(eidx_hbm, eidx_v)
    pltpu.sync_copy(ew_hbm, ew_v)

    evs = [eidx_v[pl.ds(c * L, L)] for c in range(4)]
    wvs = [ew_v[pl.ds(c * L, L)] for c in range(4)]
    bufs = [rows_a, rows_b]
    sems = [sem_a, sem_b]

    def issue(c):
        buf, sem = bufs[c % 2], sems[c % 2]
        return [pltpu.async_copy(
            act_hbm.at[h, :, pl.ds((evs[c][ll] >> 7) * 128, 128)],
            buf.at[ll], sem) for ll in range(L)]

    acc = jnp.zeros((L,), jnp.float32)
    pend = issue(0)
    for c in range(4):
        nxt = issue(c + 1) if c < 3 else []
        for cp in pend:
            cp.wait()
        buf = bufs[c % 2]
        for ll in range(L):
            col = plsc.load_gather(
                buf, [jnp.full((L,), ll, jnp.int32), lanes,
                      jnp.full((L,), evs[c][ll] & 127, jnp.int32)])
            acc = acc + jnp.full((L,), wvs[c][ll], jnp.float32) * col
        pend = nxt

    out_stage[pl.ds(0, L)] = acc
    pltpu.sync_copy(out_stage, out_hbm.at[pl.ds(h * 128, 128)])


_gather_call = functools.partial(
    pl.kernel,
    mesh=_MESH,
    out_type=jax.ShapeDtypeStruct((H * 128,), jnp.float32),
    compiler_params=pltpu.CompilerParams(needs_layout_passes=False),
    scratch_types=[
        pltpu.VMEM((128,), jnp.int32),
        pltpu.VMEM((128,), jnp.float32),
        pltpu.VMEM((L, A, 128), jnp.float32),
        pltpu.VMEM((L, A, 128), jnp.float32),
        pltpu.VMEM((128,), jnp.float32),
        pltpu.SemaphoreType.DMA,
        pltpu.SemaphoreType.DMA,
    ],
)(_gather_body)


def kernel(value, actions, mean, k):
    tf_arr = pl.pallas_call(
        _thresh_body,
        out_shape=jax.ShapeDtypeStruct((8, 128), jnp.float32),
    )(value.reshape(256, 128))
    eidx, ew = _sel_call(value.reshape(N), tf_arr.reshape(1024))
    wmean = _gather_call(actions.transpose(0, 2, 1), eidx, ew)
    return MOMENTUM * mean + wmean.reshape(H, 128)[:, :A]
